# Initial kernel scaffold; baseline (speedup 1.0000x reference)
import jax
import jax.numpy as jnp
from jax.experimental import pallas as pl


def kernel(x, edge_index, W1, b1, W2, b2, Wf1, bf1, Wf2, bf2):
    raise NotImplementedError("write your pallas kernel here")


# speedup vs baseline: 24.1619x; 24.1619x over previous
import functools

import jax
import jax.numpy as jnp
from jax import lax
from jax.experimental import pallas as pl
from jax.experimental.pallas import tpu as pltpu
from jax.experimental.pallas import tpu_sc as plsc

N = 10000
E = 320000
F = 128
H = 32

NC = 2
NS = 16
NW = NC * NS
EW = E // NW
B = 128
NB, TAIL = divmod(EW, B)
CH = 640
LAST = N - (NS - 1) * CH

_mesh = plsc.VectorSubcoreMesh(core_axis_name="c", subcore_axis_name="s")


def _rows_copy(sid, pairs):
    r0 = pl.multiple_of(sid * CH, 8)

    @pl.when(sid < NS - 1)
    def _():
        for s, d in pairs:
            pltpu.sync_copy(s.at[pl.ds(r0, CH)], d.at[pl.ds(r0, CH)])

    @pl.when(sid == NS - 1)
    def _():
        for s, d in pairs:
            pltpu.sync_copy(s.at[pl.ds(N - LAST, LAST)],
                            d.at[pl.ds(N - LAST, LAST)])


def _deg_body(dst_hbm, ones_hbm, zeros_hbm, out_hbm,
              a_sh, ones_v, ones_t, idx_v, idx_t):
    cid = lax.axis_index("c")
    sid = lax.axis_index("s")
    base = (cid * NS + sid) * EW

    pltpu.sync_copy(ones_hbm, ones_v)
    pltpu.sync_copy(ones_hbm.at[pl.ds(0, TAIL)], ones_t)
    _rows_copy(sid, [(zeros_hbm, a_sh)])
    plsc.subcore_barrier()

    def step(j, carry):
        off = pl.multiple_of(base + j * B, B)
        pltpu.sync_copy(dst_hbm.at[pl.ds(off, B)], idx_v)
        pltpu.sync_copy(ones_v, a_sh.at[idx_v], add=True)
        return carry

    lax.fori_loop(0, NB, step, 0, unroll=False)
    offt = base + NB * B
    pltpu.sync_copy(dst_hbm.at[pl.ds(offt, TAIL)], idx_t)
    pltpu.sync_copy(ones_t, a_sh.at[idx_t], add=True)

    plsc.subcore_barrier()
    _rows_copy(sid, [(a_sh, out_hbm.at[cid])])


_sc_degree = functools.partial(
    pl.kernel,
    out_type=jax.ShapeDtypeStruct((NC, N, 8), jnp.float32),
    mesh=_mesh,
    scratch_types=[
        pltpu.VMEM_SHARED((N, 8), jnp.float32),
        pltpu.VMEM((B, 8), jnp.float32),
        pltpu.VMEM((TAIL, 8), jnp.float32),
        pltpu.VMEM((B,), jnp.int32),
        pltpu.VMEM((TAIL,), jnp.int32),
    ],
)(_deg_body)


def _scat_body(g_hbm, src_hbm, dst_hbm, zeros_hbm, out_hbm,
               g_sh, a_sh, idx_s, idx_d, idx_st, idx_dt, rows, rows_t):
    cid = lax.axis_index("c")
    sid = lax.axis_index("s")
    base = (cid * NS + sid) * EW

    _rows_copy(sid, [(g_hbm, g_sh), (zeros_hbm, a_sh)])
    plsc.subcore_barrier()

    def step(j, carry):
        off = pl.multiple_of(base + j * B, B)
        pltpu.sync_copy(src_hbm.at[pl.ds(off, B)], idx_s)
        pltpu.sync_copy(dst_hbm.at[pl.ds(off, B)], idx_d)
        pltpu.sync_copy(g_sh.at[idx_s], rows)
        pltpu.sync_copy(rows, a_sh.at[idx_d], add=True)
        return carry

    lax.fori_loop(0, NB, step, 0, unroll=False)
    offt = base + NB * B
    pltpu.sync_copy(src_hbm.at[pl.ds(offt, TAIL)], idx_st)
    pltpu.sync_copy(dst_hbm.at[pl.ds(offt, TAIL)], idx_dt)
    pltpu.sync_copy(g_sh.at[idx_st], rows_t)
    pltpu.sync_copy(rows_t, a_sh.at[idx_dt], add=True)

    plsc.subcore_barrier()
    _rows_copy(sid, [(a_sh, out_hbm.at[cid])])


_sc_scatter = functools.partial(
    pl.kernel,
    out_type=jax.ShapeDtypeStruct((NC, N, H), jnp.float32),
    mesh=_mesh,
    scratch_types=[
        pltpu.VMEM_SHARED((N, H), jnp.float32),
        pltpu.VMEM_SHARED((N, H), jnp.float32),
        pltpu.VMEM((B,), jnp.int32),
        pltpu.VMEM((B,), jnp.int32),
        pltpu.VMEM((TAIL,), jnp.int32),
        pltpu.VMEM((TAIL,), jnp.int32),
        pltpu.VMEM((B, H), jnp.float32),
        pltpu.VMEM((TAIL, H), jnp.float32),
    ],
)(_scat_body)


BN = 1000
GRID = N // BN


def _dinv(dp_ref):
    d = dp_ref[0, :, 0:1] + dp_ref[1, :, 0:1] + 1.0
    return lax.rsqrt(d)


def _ab_body(x_ref, w_ref, dp_ref, o_ref):
    t = jnp.dot(x_ref[...], w_ref[...], preferred_element_type=jnp.float32)
    o_ref[...] = t * _dinv(dp_ref)


def _tc_first(x, W1, dp):
    return pl.pallas_call(
        _ab_body,
        grid=(GRID,),
        in_specs=[
            pl.BlockSpec((BN, F), lambda i: (i, 0)),
            pl.BlockSpec((F, H), lambda i: (0, 0)),
            pl.BlockSpec((NC, BN, 8), lambda i: (0, i, 0)),
        ],
        out_specs=pl.BlockSpec((BN, H), lambda i: (i, 0)),
        out_shape=jax.ShapeDtypeStruct((N, H), jnp.float32),
    )(x, W1, dp)


def _mid_body(a_ref, g_ref, dp_ref, b_ref, w_ref, o_ref):
    dinv = _dinv(dp_ref)
    h = jnp.maximum(dinv * (a_ref[0] + a_ref[1] + g_ref[...]) + b_ref[...], 0.0)
    o_ref[...] = jnp.dot(h, w_ref[...], preferred_element_type=jnp.float32) * dinv


def _tc_mid(a1, g1, dp, b1r, W2):
    return pl.pallas_call(
        _mid_body,
        grid=(GRID,),
        in_specs=[
            pl.BlockSpec((NC, BN, H), lambda i: (0, i, 0)),
            pl.BlockSpec((BN, H), lambda i: (i, 0)),
            pl.BlockSpec((NC, BN, 8), lambda i: (0, i, 0)),
            pl.BlockSpec((1, H), lambda i: (0, 0)),
            pl.BlockSpec((H, H), lambda i: (0, 0)),
        ],
        out_specs=pl.BlockSpec((BN, H), lambda i: (i, 0)),
        out_shape=jax.ShapeDtypeStruct((N, H), jnp.float32),
    )(a1, g1, dp, b1r, W2)


def _fin_body(a_ref, g_ref, dp_ref, b_ref, wf1_ref, bf1_ref, wf2_ref, bf2_ref,
              o_ref, acc):
    i = pl.program_id(0)
    dinv = _dinv(dp_ref)
    h = jnp.maximum(dinv * (a_ref[0] + a_ref[1] + g_ref[...]) + b_ref[...], 0.0)
    part = jnp.sum(h, axis=0, keepdims=True)

    @pl.when(i == 0)
    def _():
        acc[...] = part

    @pl.when(i > 0)
    def _():
        acc[...] = acc[...] + part

    @pl.when(i == GRID - 1)
    def _():
        h3 = jnp.maximum(
            jnp.dot(acc[...], wf1_ref[...], preferred_element_type=jnp.float32)
            + bf1_ref[...], 0.0)
        z = jnp.dot(h3, wf2_ref[...], preferred_element_type=jnp.float32) \
            + bf2_ref[...]
        o_ref[...] = jax.nn.sigmoid(z)


def _tc_final(a2, g2, dp, b2r, Wf1, bf1r, Wf2, bf2r):
    return pl.pallas_call(
        _fin_body,
        grid=(GRID,),
        in_specs=[
            pl.BlockSpec((NC, BN, H), lambda i: (0, i, 0)),
            pl.BlockSpec((BN, H), lambda i: (i, 0)),
            pl.BlockSpec((NC, BN, 8), lambda i: (0, i, 0)),
            pl.BlockSpec((1, H), lambda i: (0, 0)),
            pl.BlockSpec((H, 512), lambda i: (0, 0)),
            pl.BlockSpec((1, 512), lambda i: (0, 0)),
            pl.BlockSpec((512, 1), lambda i: (0, 0)),
            pl.BlockSpec((1, 1), lambda i: (0, 0)),
        ],
        out_specs=pl.BlockSpec((1, 1), lambda i: (0, 0)),
        out_shape=jax.ShapeDtypeStruct((1, 1), jnp.float32),
        scratch_shapes=[pltpu.VMEM((1, H), jnp.float32)],
    )(a2, g2, dp, b2r, Wf1, bf1r, Wf2, bf2r)


def kernel(x, edge_index, W1, b1, W2, b2, Wf1, bf1, Wf2, bf2):
    src = edge_index[0]
    dst = edge_index[1]
    zeros8 = jnp.zeros((N, 8), jnp.float32)
    ones8 = jnp.ones((B, 8), jnp.float32)
    zerosh = jnp.zeros((N, H), jnp.float32)

    dp = _sc_degree(dst, ones8, zeros8)
    g1 = _tc_first(x, W1, dp)
    a1 = _sc_scatter(g1, src, dst, zerosh)
    g2 = _tc_mid(a1, g1, dp, b1.reshape(1, H), W2)
    a2 = _sc_scatter(g2, src, dst, zerosh)
    return _tc_final(a2, g2, dp, b2.reshape(1, H), Wf1,
                     bf1.reshape(1, 512), Wf2, bf2.reshape(1, 1))


# speedup vs baseline: 38.5673x; 1.5962x over previous
import functools

import jax
import jax.numpy as jnp
from jax import lax
from jax.experimental import pallas as pl
from jax.experimental.pallas import tpu as pltpu
from jax.experimental.pallas import tpu_sc as plsc

N = 10000
E = 320000
F = 128
H = 32

NC = 2
NS = 16
NW = NC * NS
EW = E // NW
B = 128
NB, TAIL = divmod(EW, B)
CH = 640
LAST = N - (NS - 1) * CH

_mesh = plsc.VectorSubcoreMesh(core_axis_name="c", subcore_axis_name="s")


def _rows_copy(sid, pairs):
    r0 = pl.multiple_of(sid * CH, 8)

    @pl.when(sid < NS - 1)
    def _():
        for s, d in pairs:
            pltpu.sync_copy(s.at[pl.ds(r0, CH)], d.at[pl.ds(r0, CH)])

    @pl.when(sid == NS - 1)
    def _():
        for s, d in pairs:
            pltpu.sync_copy(s.at[pl.ds(N - LAST, LAST)],
                            d.at[pl.ds(N - LAST, LAST)])


KS = 3
NG = NB // KS


def _deg_body(dst_hbm, ones_hbm, zeros_hbm, out_hbm,
              a_sh, ones_v, ones_t, idx2, idx_t, sem_i, sem_s):
    cid = lax.axis_index("c")
    sid = lax.axis_index("s")
    base = (cid * NS + sid) * EW

    pltpu.sync_copy(ones_hbm, ones_v)
    pltpu.sync_copy(ones_hbm.at[pl.ds(0, TAIL)], ones_t)
    _rows_copy(sid, [(zeros_hbm, a_sh)])
    plsc.subcore_barrier()

    def group(gi, carry):
        g0 = base + gi * (KS * B)
        ics = [pltpu.async_copy(
                   dst_hbm.at[pl.ds(pl.multiple_of(g0 + k * B, B), B)],
                   idx2.at[k], sem_i) for k in range(KS)]
        for c in ics:
            c.wait()
        scs = [pltpu.async_copy(ones_v, a_sh.at[idx2.at[k]], sem_s, add=True)
               for k in range(KS)]
        for c in scs:
            c.wait()
        return carry

    lax.fori_loop(0, NG, group, 0, unroll=False)
    offt = base + NB * B
    pltpu.sync_copy(dst_hbm.at[pl.ds(offt, TAIL)], idx_t)
    pltpu.sync_copy(ones_t, a_sh.at[idx_t], add=True)

    plsc.subcore_barrier()
    _rows_copy(sid, [(a_sh, out_hbm.at[cid])])


_sc_degree = functools.partial(
    pl.kernel,
    out_type=jax.ShapeDtypeStruct((NC, N, 8), jnp.float32),
    mesh=_mesh,
    scratch_types=[
        pltpu.VMEM_SHARED((N, 8), jnp.float32),
        pltpu.VMEM((B, 8), jnp.float32),
        pltpu.VMEM((TAIL, 8), jnp.float32),
        pltpu.VMEM((KS, B), jnp.int32),
        pltpu.VMEM((TAIL,), jnp.int32),
        pltpu.SemaphoreType.DMA,
        pltpu.SemaphoreType.DMA,
    ],
)(_deg_body)


def _scat_body(g_hbm, src_hbm, dst_hbm, zeros_hbm, out_hbm,
               g_sh, a_sh, idx_s2, idx_d2, idx_st, idx_dt, rows3, rows_t,
               sem_i, sem_g, sem_s):
    cid = lax.axis_index("c")
    sid = lax.axis_index("s")
    base = (cid * NS + sid) * EW

    _rows_copy(sid, [(g_hbm, g_sh), (zeros_hbm, a_sh)])
    plsc.subcore_barrier()

    def group(gi, carry):
        g0 = base + gi * (KS * B)
        ics = []
        for k in range(KS):
            off = pl.multiple_of(g0 + k * B, B)
            ics.append(pltpu.async_copy(src_hbm.at[pl.ds(off, B)],
                                        idx_s2.at[k], sem_i))
            ics.append(pltpu.async_copy(dst_hbm.at[pl.ds(off, B)],
                                        idx_d2.at[k], sem_i))
        for c in ics:
            c.wait()
        gcs = [pltpu.async_copy(g_sh.at[idx_s2.at[k]], rows3.at[k], sem_g)
               for k in range(KS)]
        for c in gcs:
            c.wait()
        scs = [pltpu.async_copy(rows3.at[k], a_sh.at[idx_d2.at[k]], sem_s,
                                add=True) for k in range(KS)]
        for c in scs:
            c.wait()
        return carry

    lax.fori_loop(0, NG, group, 0, unroll=False)
    offt = base + NB * B
    pltpu.sync_copy(src_hbm.at[pl.ds(offt, TAIL)], idx_st)
    pltpu.sync_copy(dst_hbm.at[pl.ds(offt, TAIL)], idx_dt)
    pltpu.sync_copy(g_sh.at[idx_st], rows_t)
    pltpu.sync_copy(rows_t, a_sh.at[idx_dt], add=True)

    plsc.subcore_barrier()
    _rows_copy(sid, [(a_sh, out_hbm.at[cid])])


_sc_scatter = functools.partial(
    pl.kernel,
    out_type=jax.ShapeDtypeStruct((NC, N, H), jnp.float32),
    mesh=_mesh,
    scratch_types=[
        pltpu.VMEM_SHARED((N, H), jnp.float32),
        pltpu.VMEM_SHARED((N, H), jnp.float32),
        pltpu.VMEM((KS, B), jnp.int32),
        pltpu.VMEM((KS, B), jnp.int32),
        pltpu.VMEM((TAIL,), jnp.int32),
        pltpu.VMEM((TAIL,), jnp.int32),
        pltpu.VMEM((KS, B, H), jnp.float32),
        pltpu.VMEM((TAIL, H), jnp.float32),
        pltpu.SemaphoreType.DMA,
        pltpu.SemaphoreType.DMA,
        pltpu.SemaphoreType.DMA,
    ],
)(_scat_body)


BN = 1000
GRID = N // BN


def _dinv(dp_ref):
    d = dp_ref[0, :, 0:1] + dp_ref[1, :, 0:1] + 1.0
    return lax.rsqrt(d)


def _ab_body(x_ref, w_ref, dp_ref, o_ref):
    t = jnp.dot(x_ref[...], w_ref[...], preferred_element_type=jnp.float32)
    o_ref[...] = t * _dinv(dp_ref)


def _tc_first(x, W1, dp):
    return pl.pallas_call(
        _ab_body,
        grid=(GRID,),
        in_specs=[
            pl.BlockSpec((BN, F), lambda i: (i, 0)),
            pl.BlockSpec((F, H), lambda i: (0, 0)),
            pl.BlockSpec((NC, BN, 8), lambda i: (0, i, 0)),
        ],
        out_specs=pl.BlockSpec((BN, H), lambda i: (i, 0)),
        out_shape=jax.ShapeDtypeStruct((N, H), jnp.float32),
    )(x, W1, dp)


def _mid_body(a_ref, g_ref, dp_ref, b_ref, w_ref, o_ref):
    dinv = _dinv(dp_ref)
    h = jnp.maximum(dinv * (a_ref[0] + a_ref[1] + g_ref[...]) + b_ref[...], 0.0)
    o_ref[...] = jnp.dot(h, w_ref[...], preferred_element_type=jnp.float32) * dinv


def _tc_mid(a1, g1, dp, b1r, W2):
    return pl.pallas_call(
        _mid_body,
        grid=(GRID,),
        in_specs=[
            pl.BlockSpec((NC, BN, H), lambda i: (0, i, 0)),
            pl.BlockSpec((BN, H), lambda i: (i, 0)),
            pl.BlockSpec((NC, BN, 8), lambda i: (0, i, 0)),
            pl.BlockSpec((1, H), lambda i: (0, 0)),
            pl.BlockSpec((H, H), lambda i: (0, 0)),
        ],
        out_specs=pl.BlockSpec((BN, H), lambda i: (i, 0)),
        out_shape=jax.ShapeDtypeStruct((N, H), jnp.float32),
    )(a1, g1, dp, b1r, W2)


def _fin_body(a_ref, g_ref, dp_ref, b_ref, wf1_ref, bf1_ref, wf2_ref, bf2_ref,
              o_ref, acc):
    i = pl.program_id(0)
    dinv = _dinv(dp_ref)
    h = jnp.maximum(dinv * (a_ref[0] + a_ref[1] + g_ref[...]) + b_ref[...], 0.0)
    part = jnp.sum(h, axis=0, keepdims=True)

    @pl.when(i == 0)
    def _():
        acc[...] = part

    @pl.when(i > 0)
    def _():
        acc[...] = acc[...] + part

    @pl.when(i == GRID - 1)
    def _():
        h3 = jnp.maximum(
            jnp.dot(acc[...], wf1_ref[...], preferred_element_type=jnp.float32)
            + bf1_ref[...], 0.0)
        z = jnp.dot(h3, wf2_ref[...], preferred_element_type=jnp.float32) \
            + bf2_ref[...]
        o_ref[...] = jax.nn.sigmoid(z)


def _tc_final(a2, g2, dp, b2r, Wf1, bf1r, Wf2, bf2r):
    return pl.pallas_call(
        _fin_body,
        grid=(GRID,),
        in_specs=[
            pl.BlockSpec((NC, BN, H), lambda i: (0, i, 0)),
            pl.BlockSpec((BN, H), lambda i: (i, 0)),
            pl.BlockSpec((NC, BN, 8), lambda i: (0, i, 0)),
            pl.BlockSpec((1, H), lambda i: (0, 0)),
            pl.BlockSpec((H, 512), lambda i: (0, 0)),
            pl.BlockSpec((1, 512), lambda i: (0, 0)),
            pl.BlockSpec((512, 1), lambda i: (0, 0)),
            pl.BlockSpec((1, 1), lambda i: (0, 0)),
        ],
        out_specs=pl.BlockSpec((1, 1), lambda i: (0, 0)),
        out_shape=jax.ShapeDtypeStruct((1, 1), jnp.float32),
        scratch_shapes=[pltpu.VMEM((1, H), jnp.float32)],
    )(a2, g2, dp, b2r, Wf1, bf1r, Wf2, bf2r)


def kernel(x, edge_index, W1, b1, W2, b2, Wf1, bf1, Wf2, bf2):
    src = edge_index[0]
    dst = edge_index[1]
    zeros8 = jnp.zeros((N, 8), jnp.float32)
    ones8 = jnp.ones((B, 8), jnp.float32)
    zerosh = jnp.zeros((N, H), jnp.float32)

    dp = _sc_degree(dst, ones8, zeros8)
    g1 = _tc_first(x, W1, dp)
    a1 = _sc_scatter(g1, src, dst, zerosh)
    g2 = _tc_mid(a1, g1, dp, b1.reshape(1, H), W2)
    a2 = _sc_scatter(g2, src, dst, zerosh)
    return _tc_final(a2, g2, dp, b2.reshape(1, H), Wf1,
                     bf1.reshape(1, 512), Wf2, bf2.reshape(1, 1))


# speedup vs baseline: 41.9870x; 1.0887x over previous
import functools

import jax
import jax.numpy as jnp
from jax import lax
from jax.experimental import pallas as pl
from jax.experimental.pallas import tpu as pltpu
from jax.experimental.pallas import tpu_sc as plsc

N = 10000
E = 320000
F = 128
H = 32

NC = 2
NS = 16
NW = NC * NS
EW = E // NW
B = 128
NB, TAIL = divmod(EW, B)
CH = 640
LAST = N - (NS - 1) * CH

_mesh = plsc.VectorSubcoreMesh(core_axis_name="c", subcore_axis_name="s")


def _rows_copy(sid, pairs):
    r0 = pl.multiple_of(sid * CH, 8)

    @pl.when(sid < NS - 1)
    def _():
        for s, d in pairs:
            pltpu.sync_copy(s.at[pl.ds(r0, CH)], d.at[pl.ds(r0, CH)])

    @pl.when(sid == NS - 1)
    def _():
        for s, d in pairs:
            pltpu.sync_copy(s.at[pl.ds(N - LAST, LAST)],
                            d.at[pl.ds(N - LAST, LAST)])


KD = 8
KG = 4


def _deg_body(dst_hbm, ones_hbm, zeros_hbm, out_hbm,
              a_sh, ones_v, ones_t, idx8, idx_t, *sems):
    sem_i, sem_s = sems[:KD], sems[KD:]
    cid = lax.axis_index("c")
    sid = lax.axis_index("s")
    base = (cid * NS + sid) * EW

    pltpu.sync_copy(ones_hbm, ones_v)
    pltpu.sync_copy(ones_hbm.at[pl.ds(0, TAIL)], ones_t)
    _rows_copy(sid, [(zeros_hbm, a_sh)])
    plsc.subcore_barrier()

    def group(nb, g0):
        ics = [pltpu.async_copy(
                   dst_hbm.at[pl.ds(pl.multiple_of(g0 + k * B, B), B)],
                   idx8.at[k], sem_i[k]) for k in range(nb)]
        for c in ics:
            c.wait()
        scs = [pltpu.async_copy(ones_v, a_sh.at[idx8.at[k]],
                                sem_s[k], add=True) for k in range(nb)]
        for c in scs:
            c.wait()

    def body(gi, carry):
        group(KD, base + gi * (KD * B))
        return carry

    lax.fori_loop(0, NB // KD, body, 0, unroll=False)
    group(NB % KD, base + (NB - NB % KD) * B)
    offt = base + NB * B
    pltpu.sync_copy(dst_hbm.at[pl.ds(offt, TAIL)], idx_t)
    pltpu.sync_copy(ones_t, a_sh.at[idx_t], add=True)

    plsc.subcore_barrier()
    _rows_copy(sid, [(a_sh, out_hbm.at[cid])])


_sc_degree = functools.partial(
    pl.kernel,
    out_type=jax.ShapeDtypeStruct((NC, N, 8), jnp.float32),
    mesh=_mesh,
    scratch_types=[
        pltpu.VMEM_SHARED((N, 8), jnp.float32),
        pltpu.VMEM((B, 8), jnp.float32),
        pltpu.VMEM((TAIL, 8), jnp.float32),
        pltpu.VMEM((KD, B), jnp.int32),
        pltpu.VMEM((TAIL,), jnp.int32),
    ] + [pltpu.SemaphoreType.DMA] * (2 * KD),
)(_deg_body)


def _scat_body(g_hbm, src_hbm, dst_hbm, zeros_hbm, out_hbm,
               g_sh, a_sh, idx_s8, idx_d8, rows4, idx_st, idx_dt, rows_t,
               *sems):
    sem_i, sem_g, sem_s = sems[:KG], sems[KG:2 * KG], sems[2 * KG:]
    cid = lax.axis_index("c")
    sid = lax.axis_index("s")
    base = (cid * NS + sid) * EW

    _rows_copy(sid, [(g_hbm, g_sh), (zeros_hbm, a_sh)])
    plsc.subcore_barrier()

    def group(nb, g0):
        ics = []
        for k in range(nb):
            off = pl.multiple_of(g0 + k * B, B)
            ics.append(pltpu.async_copy(src_hbm.at[pl.ds(off, B)],
                                        idx_s8.at[k], sem_i[k]))
            ics.append(pltpu.async_copy(dst_hbm.at[pl.ds(off, B)],
                                        idx_d8.at[k], sem_i[k]))
        gcs = []
        for k in range(nb):
            ics[2 * k].wait()
            ics[2 * k + 1].wait()
            gcs.append(pltpu.async_copy(g_sh.at[idx_s8.at[k]], rows4.at[k],
                                        sem_g[k]))
        for c in gcs:
            c.wait()
        scs = [pltpu.async_copy(rows4.at[k], a_sh.at[idx_d8.at[k]],
                                sem_s[k], add=True) for k in range(nb)]
        for c in scs:
            c.wait()

    def body(gi, carry):
        group(KG, base + gi * (KG * B))
        return carry

    lax.fori_loop(0, NB // KG, body, 0, unroll=False)
    group(NB % KG, base + (NB - NB % KG) * B)
    offt = base + NB * B
    pltpu.sync_copy(src_hbm.at[pl.ds(offt, TAIL)], idx_st)
    pltpu.sync_copy(dst_hbm.at[pl.ds(offt, TAIL)], idx_dt)
    pltpu.sync_copy(g_sh.at[idx_st], rows_t)
    pltpu.sync_copy(rows_t, a_sh.at[idx_dt], add=True)

    plsc.subcore_barrier()
    _rows_copy(sid, [(a_sh, out_hbm.at[cid])])


_sc_scatter = functools.partial(
    pl.kernel,
    out_type=jax.ShapeDtypeStruct((NC, N, H), jnp.float32),
    mesh=_mesh,
    scratch_types=[
        pltpu.VMEM_SHARED((N, H), jnp.float32),
        pltpu.VMEM_SHARED((N, H), jnp.float32),
        pltpu.VMEM((KG, B), jnp.int32),
        pltpu.VMEM((KG, B), jnp.int32),
        pltpu.VMEM((KG, B, H), jnp.float32),
        pltpu.VMEM((TAIL,), jnp.int32),
        pltpu.VMEM((TAIL,), jnp.int32),
        pltpu.VMEM((TAIL, H), jnp.float32),
    ] + [pltpu.SemaphoreType.DMA] * (3 * KG),
)(_scat_body)


BN = 1000
GRID = N // BN


def _dinv(dp_ref):
    d = dp_ref[0, :, 0:1] + dp_ref[1, :, 0:1] + 1.0
    return lax.rsqrt(d)


def _ab_body(x_ref, w_ref, dp_ref, o_ref):
    t = jnp.dot(x_ref[...], w_ref[...], preferred_element_type=jnp.float32)
    o_ref[...] = t * _dinv(dp_ref)


def _tc_first(x, W1, dp):
    return pl.pallas_call(
        _ab_body,
        grid=(GRID,),
        in_specs=[
            pl.BlockSpec((BN, F), lambda i: (i, 0)),
            pl.BlockSpec((F, H), lambda i: (0, 0)),
            pl.BlockSpec((NC, BN, 8), lambda i: (0, i, 0)),
        ],
        out_specs=pl.BlockSpec((BN, H), lambda i: (i, 0)),
        out_shape=jax.ShapeDtypeStruct((N, H), jnp.float32),
    )(x, W1, dp)


def _mid_body(a_ref, g_ref, dp_ref, b_ref, w_ref, o_ref):
    dinv = _dinv(dp_ref)
    h = jnp.maximum(dinv * (a_ref[0] + a_ref[1] + g_ref[...]) + b_ref[...], 0.0)
    o_ref[...] = jnp.dot(h, w_ref[...], preferred_element_type=jnp.float32) * dinv


def _tc_mid(a1, g1, dp, b1r, W2):
    return pl.pallas_call(
        _mid_body,
        grid=(GRID,),
        in_specs=[
            pl.BlockSpec((NC, BN, H), lambda i: (0, i, 0)),
            pl.BlockSpec((BN, H), lambda i: (i, 0)),
            pl.BlockSpec((NC, BN, 8), lambda i: (0, i, 0)),
            pl.BlockSpec((1, H), lambda i: (0, 0)),
            pl.BlockSpec((H, H), lambda i: (0, 0)),
        ],
        out_specs=pl.BlockSpec((BN, H), lambda i: (i, 0)),
        out_shape=jax.ShapeDtypeStruct((N, H), jnp.float32),
    )(a1, g1, dp, b1r, W2)


def _fin_body(a_ref, g_ref, dp_ref, b_ref, wf1_ref, bf1_ref, wf2_ref, bf2_ref,
              o_ref, acc):
    i = pl.program_id(0)
    dinv = _dinv(dp_ref)
    h = jnp.maximum(dinv * (a_ref[0] + a_ref[1] + g_ref[...]) + b_ref[...], 0.0)
    part = jnp.sum(h, axis=0, keepdims=True)

    @pl.when(i == 0)
    def _():
        acc[...] = part

    @pl.when(i > 0)
    def _():
        acc[...] = acc[...] + part

    @pl.when(i == GRID - 1)
    def _():
        h3 = jnp.maximum(
            jnp.dot(acc[...], wf1_ref[...], preferred_element_type=jnp.float32)
            + bf1_ref[...], 0.0)
        z = jnp.dot(h3, wf2_ref[...], preferred_element_type=jnp.float32) \
            + bf2_ref[...]
        o_ref[...] = jax.nn.sigmoid(z)


def _tc_final(a2, g2, dp, b2r, Wf1, bf1r, Wf2, bf2r):
    return pl.pallas_call(
        _fin_body,
        grid=(GRID,),
        in_specs=[
            pl.BlockSpec((NC, BN, H), lambda i: (0, i, 0)),
            pl.BlockSpec((BN, H), lambda i: (i, 0)),
            pl.BlockSpec((NC, BN, 8), lambda i: (0, i, 0)),
            pl.BlockSpec((1, H), lambda i: (0, 0)),
            pl.BlockSpec((H, 512), lambda i: (0, 0)),
            pl.BlockSpec((1, 512), lambda i: (0, 0)),
            pl.BlockSpec((512, 1), lambda i: (0, 0)),
            pl.BlockSpec((1, 1), lambda i: (0, 0)),
        ],
        out_specs=pl.BlockSpec((1, 1), lambda i: (0, 0)),
        out_shape=jax.ShapeDtypeStruct((1, 1), jnp.float32),
        scratch_shapes=[pltpu.VMEM((1, H), jnp.float32)],
    )(a2, g2, dp, b2r, Wf1, bf1r, Wf2, bf2r)


def kernel(x, edge_index, W1, b1, W2, b2, Wf1, bf1, Wf2, bf2):
    src = edge_index[0]
    dst = edge_index[1]
    zeros8 = jnp.zeros((N, 8), jnp.float32)
    ones8 = jnp.ones((B, 8), jnp.float32)
    zerosh = jnp.zeros((N, H), jnp.float32)

    dp = _sc_degree(dst, ones8, zeros8)
    g1 = _tc_first(x, W1, dp)
    a1 = _sc_scatter(g1, src, dst, zerosh)
    g2 = _tc_mid(a1, g1, dp, b1.reshape(1, H), W2)
    a2 = _sc_scatter(g2, src, dst, zerosh)
    return _tc_final(a2, g2, dp, b2.reshape(1, H), Wf1,
                     bf1.reshape(1, 512), Wf2, bf2.reshape(1, 1))


# speedup vs baseline: 43.3007x; 1.0313x over previous
import functools

import jax
import jax.numpy as jnp
from jax import lax
from jax.experimental import pallas as pl
from jax.experimental.pallas import tpu as pltpu
from jax.experimental.pallas import tpu_sc as plsc

N = 10000
E = 320000
F = 128
H = 32

NC = 2
NS = 16
NW = NC * NS
EW = E // NW
B = 128
NB, TAIL = divmod(EW, B)
CH = 640
LAST = N - (NS - 1) * CH

_mesh = plsc.VectorSubcoreMesh(core_axis_name="c", subcore_axis_name="s")


def _rows_copy(sid, pairs):
    r0 = pl.multiple_of(sid * CH, 8)

    @pl.when(sid < NS - 1)
    def _():
        for s, d in pairs:
            pltpu.sync_copy(s.at[pl.ds(r0, CH)], d.at[pl.ds(r0, CH)])

    @pl.when(sid == NS - 1)
    def _():
        for s, d in pairs:
            pltpu.sync_copy(s.at[pl.ds(N - LAST, LAST)],
                            d.at[pl.ds(N - LAST, LAST)])


KD = 12
KG = 5


def _deg_body(dst_hbm, ones_hbm, zeros_hbm, out_hbm,
              a_sh, ones_v, ones_t, idx8, idx_t, *sems):
    sem_i, sem_s = sems[:KD], sems[KD:]
    cid = lax.axis_index("c")
    sid = lax.axis_index("s")
    base = (cid * NS + sid) * EW

    pltpu.sync_copy(ones_hbm, ones_v)
    pltpu.sync_copy(ones_hbm.at[pl.ds(0, TAIL)], ones_t)
    _rows_copy(sid, [(zeros_hbm, a_sh)])
    plsc.subcore_barrier()

    def group(nb, g0):
        ics = [pltpu.async_copy(
                   dst_hbm.at[pl.ds(pl.multiple_of(g0 + k * B, B), B)],
                   idx8.at[k], sem_i[k]) for k in range(nb)]
        for c in ics:
            c.wait()
        scs = [pltpu.async_copy(ones_v, a_sh.at[idx8.at[k]],
                                sem_s[k], add=True) for k in range(nb)]
        for c in scs:
            c.wait()

    def body(gi, carry):
        group(KD, base + gi * (KD * B))
        return carry

    lax.fori_loop(0, NB // KD, body, 0, unroll=False)
    group(NB % KD, base + (NB - NB % KD) * B)
    offt = base + NB * B
    pltpu.sync_copy(dst_hbm.at[pl.ds(offt, TAIL)], idx_t)
    pltpu.sync_copy(ones_t, a_sh.at[idx_t], add=True)

    plsc.subcore_barrier()
    _rows_copy(sid, [(a_sh, out_hbm.at[cid])])


_sc_degree = functools.partial(
    pl.kernel,
    out_type=jax.ShapeDtypeStruct((NC, N, 8), jnp.float32),
    mesh=_mesh,
    scratch_types=[
        pltpu.VMEM_SHARED((N, 8), jnp.float32),
        pltpu.VMEM((B, 8), jnp.float32),
        pltpu.VMEM((TAIL, 8), jnp.float32),
        pltpu.VMEM((KD, B), jnp.int32),
        pltpu.VMEM((TAIL,), jnp.int32),
    ] + [pltpu.SemaphoreType.DMA] * (2 * KD),
)(_deg_body)


def _scat_body(g_hbm, src_hbm, dst_hbm, zeros_hbm, out_hbm,
               g_sh, a_sh, idx_s8, idx_d8, rows4, idx_st, idx_dt, rows_t,
               *sems):
    sem_i, sem_g, sem_s = sems[:KG], sems[KG:2 * KG], sems[2 * KG:]
    cid = lax.axis_index("c")
    sid = lax.axis_index("s")
    base = (cid * NS + sid) * EW

    _rows_copy(sid, [(g_hbm, g_sh), (zeros_hbm, a_sh)])
    plsc.subcore_barrier()

    def group(nb, g0):
        ics = []
        for k in range(nb):
            off = pl.multiple_of(g0 + k * B, B)
            ics.append(pltpu.async_copy(src_hbm.at[pl.ds(off, B)],
                                        idx_s8.at[k], sem_i[k]))
            ics.append(pltpu.async_copy(dst_hbm.at[pl.ds(off, B)],
                                        idx_d8.at[k], sem_i[k]))
        gcs = []
        for k in range(nb):
            ics[2 * k].wait()
            ics[2 * k + 1].wait()
            gcs.append(pltpu.async_copy(g_sh.at[idx_s8.at[k]], rows4.at[k],
                                        sem_g[k]))
        for c in gcs:
            c.wait()
        scs = [pltpu.async_copy(rows4.at[k], a_sh.at[idx_d8.at[k]],
                                sem_s[k], add=True) for k in range(nb)]
        for c in scs:
            c.wait()

    def body(gi, carry):
        group(KG, base + gi * (KG * B))
        return carry

    lax.fori_loop(0, NB // KG, body, 0, unroll=False)
    group(NB % KG, base + (NB - NB % KG) * B)
    offt = base + NB * B
    pltpu.sync_copy(src_hbm.at[pl.ds(offt, TAIL)], idx_st)
    pltpu.sync_copy(dst_hbm.at[pl.ds(offt, TAIL)], idx_dt)
    pltpu.sync_copy(g_sh.at[idx_st], rows_t)
    pltpu.sync_copy(rows_t, a_sh.at[idx_dt], add=True)

    plsc.subcore_barrier()
    _rows_copy(sid, [(a_sh, out_hbm.at[cid])])


_sc_scatter = functools.partial(
    pl.kernel,
    out_type=jax.ShapeDtypeStruct((NC, N, H), jnp.float32),
    mesh=_mesh,
    scratch_types=[
        pltpu.VMEM_SHARED((N, H), jnp.float32),
        pltpu.VMEM_SHARED((N, H), jnp.float32),
        pltpu.VMEM((KG, B), jnp.int32),
        pltpu.VMEM((KG, B), jnp.int32),
        pltpu.VMEM((KG, B, H), jnp.float32),
        pltpu.VMEM((TAIL,), jnp.int32),
        pltpu.VMEM((TAIL,), jnp.int32),
        pltpu.VMEM((TAIL, H), jnp.float32),
    ] + [pltpu.SemaphoreType.DMA] * (3 * KG),
)(_scat_body)


BN = 1000
GRID = N // BN


def _dinv(dp_ref):
    d = dp_ref[0, :, 0:1] + dp_ref[1, :, 0:1] + 1.0
    return lax.rsqrt(d)


def _mm_body(x_ref, w_ref, o_ref):
    o_ref[...] = jnp.dot(x_ref[...], w_ref[...],
                         preferred_element_type=jnp.float32)


def _tc_matmul(x, W1):
    return pl.pallas_call(
        _mm_body,
        grid=(GRID,),
        in_specs=[
            pl.BlockSpec((BN, F), lambda i: (i, 0)),
            pl.BlockSpec((F, H), lambda i: (0, 0)),
        ],
        out_specs=pl.BlockSpec((BN, H), lambda i: (i, 0)),
        out_shape=jax.ShapeDtypeStruct((N, H), jnp.float32),
    )(x, W1)


def _sc_body(t_ref, dp_ref, o_ref):
    o_ref[...] = t_ref[...] * _dinv(dp_ref)


def _tc_scale(t1, dp):
    return pl.pallas_call(
        _sc_body,
        grid=(GRID,),
        in_specs=[
            pl.BlockSpec((BN, H), lambda i: (i, 0)),
            pl.BlockSpec((NC, BN, 8), lambda i: (0, i, 0)),
        ],
        out_specs=pl.BlockSpec((BN, H), lambda i: (i, 0)),
        out_shape=jax.ShapeDtypeStruct((N, H), jnp.float32),
    )(t1, dp)


def _mid_body(a_ref, g_ref, dp_ref, b_ref, w_ref, o_ref):
    dinv = _dinv(dp_ref)
    h = jnp.maximum(dinv * (a_ref[0] + a_ref[1] + g_ref[...]) + b_ref[...], 0.0)
    o_ref[...] = jnp.dot(h, w_ref[...], preferred_element_type=jnp.float32) * dinv


def _tc_mid(a1, g1, dp, b1r, W2):
    return pl.pallas_call(
        _mid_body,
        grid=(GRID,),
        in_specs=[
            pl.BlockSpec((NC, BN, H), lambda i: (0, i, 0)),
            pl.BlockSpec((BN, H), lambda i: (i, 0)),
            pl.BlockSpec((NC, BN, 8), lambda i: (0, i, 0)),
            pl.BlockSpec((1, H), lambda i: (0, 0)),
            pl.BlockSpec((H, H), lambda i: (0, 0)),
        ],
        out_specs=pl.BlockSpec((BN, H), lambda i: (i, 0)),
        out_shape=jax.ShapeDtypeStruct((N, H), jnp.float32),
    )(a1, g1, dp, b1r, W2)


def _fin_body(a_ref, g_ref, dp_ref, b_ref, wf1_ref, bf1_ref, wf2_ref, bf2_ref,
              o_ref, acc):
    i = pl.program_id(0)
    dinv = _dinv(dp_ref)
    h = jnp.maximum(dinv * (a_ref[0] + a_ref[1] + g_ref[...]) + b_ref[...], 0.0)
    part = jnp.sum(h, axis=0, keepdims=True)

    @pl.when(i == 0)
    def _():
        acc[...] = part

    @pl.when(i > 0)
    def _():
        acc[...] = acc[...] + part

    @pl.when(i == GRID - 1)
    def _():
        h3 = jnp.maximum(
            jnp.dot(acc[...], wf1_ref[...], preferred_element_type=jnp.float32)
            + bf1_ref[...], 0.0)
        z = jnp.dot(h3, wf2_ref[...], preferred_element_type=jnp.float32) \
            + bf2_ref[...]
        o_ref[...] = jax.nn.sigmoid(z)


def _tc_final(a2, g2, dp, b2r, Wf1, bf1r, Wf2, bf2r):
    return pl.pallas_call(
        _fin_body,
        grid=(GRID,),
        in_specs=[
            pl.BlockSpec((NC, BN, H), lambda i: (0, i, 0)),
            pl.BlockSpec((BN, H), lambda i: (i, 0)),
            pl.BlockSpec((NC, BN, 8), lambda i: (0, i, 0)),
            pl.BlockSpec((1, H), lambda i: (0, 0)),
            pl.BlockSpec((H, 512), lambda i: (0, 0)),
            pl.BlockSpec((1, 512), lambda i: (0, 0)),
            pl.BlockSpec((512, 1), lambda i: (0, 0)),
            pl.BlockSpec((1, 1), lambda i: (0, 0)),
        ],
        out_specs=pl.BlockSpec((1, 1), lambda i: (0, 0)),
        out_shape=jax.ShapeDtypeStruct((1, 1), jnp.float32),
        scratch_shapes=[pltpu.VMEM((1, H), jnp.float32)],
    )(a2, g2, dp, b2r, Wf1, bf1r, Wf2, bf2r)


def kernel(x, edge_index, W1, b1, W2, b2, Wf1, bf1, Wf2, bf2):
    src = edge_index[0]
    dst = edge_index[1]
    zeros8 = jnp.zeros((N, 8), jnp.float32)
    ones8 = jnp.ones((B, 8), jnp.float32)
    zerosh = jnp.zeros((N, H), jnp.float32)

    dp = _sc_degree(dst, ones8, zeros8)
    t1 = _tc_matmul(x, W1)
    g1 = _tc_scale(t1, dp)
    a1 = _sc_scatter(g1, src, dst, zerosh)
    g2 = _tc_mid(a1, g1, dp, b1.reshape(1, H), W2)
    a2 = _sc_scatter(g2, src, dst, zerosh)
    return _tc_final(a2, g2, dp, b2.reshape(1, H), Wf1,
                     bf1.reshape(1, 512), Wf2, bf2.reshape(1, 1))
